# baseline (speedup 1.0000x reference)
import functools

import jax
import jax.numpy as jnp
from jax import lax
from jax.experimental import pallas as pl
from jax.experimental.pallas import tpu as pltpu
from jax.experimental.pallas import tpu_sc as plsc

MIN_VAL = 0
MAX_VAL = 999999
OUTPUT_DIM = 64
BATCH = 16384
IDX_CHUNK = 128

_KERNEL_CACHE = {}


def _make_gather_kernel(batch, dim):
    info = plsc.get_sparse_core_info()
    num_cores, num_subcores, lanes = (
        info.num_cores,
        info.num_subcores,
        info.num_lanes,
    )
    num_workers = num_cores * num_subcores
    b_per_w = batch // num_workers
    n_chunk = b_per_w // IDX_CHUNK
    mesh = plsc.VectorSubcoreMesh(core_axis_name="c", subcore_axis_name="s")

    @functools.partial(
        pl.kernel,
        mesh=mesh,
        compiler_params=pltpu.CompilerParams(use_tc_tiling_on_sc=False),
        out_type=jax.ShapeDtypeStruct((batch, dim), jnp.float32),
        scratch_types=[
            pltpu.VMEM((n_chunk, IDX_CHUNK), jnp.int32),
            pltpu.VMEM((b_per_w, dim), jnp.float32),
            pltpu.SemaphoreType.DMA,
        ],
    )
    def gather_kernel(ints_hbm, table_hbm, out_hbm, idx_v, rows_v, sem):
        wid = lax.axis_index("s") * num_cores + lax.axis_index("c")
        base = wid * b_per_w
        copies = []
        for j in range(n_chunk):
            row = idx_v.at[j]
            pltpu.sync_copy(ints_hbm.at[pl.ds(base + j * IDX_CHUNK, IDX_CHUNK)], row)
            for t in range(IDX_CHUNK // lanes):
                v = row[pl.ds(t * lanes, lanes)]
                row[pl.ds(t * lanes, lanes)] = (
                    jnp.minimum(jnp.maximum(v, MIN_VAL), MAX_VAL) - MIN_VAL
                )
            copies.append(
                pltpu.async_copy(
                    table_hbm.at[row],
                    rows_v.at[pl.ds(j * IDX_CHUNK, IDX_CHUNK)],
                    sem,
                )
            )
        for c in copies:
            c.wait()
        pltpu.sync_copy(rows_v, out_hbm.at[pl.ds(base, b_per_w)])

    return gather_kernel


def kernel(ints, table):
    batch = ints.shape[0]
    dim = table.shape[1]
    key = (batch, dim)
    if key not in _KERNEL_CACHE:
        _KERNEL_CACHE[key] = _make_gather_kernel(batch, dim)
    out = _KERNEL_CACHE[key](ints.astype(jnp.int32), table)
    mask = jnp.ones((batch, 1), dtype=jnp.float32)
    return out[:, None, :], mask


# speedup vs baseline: 1.7161x; 1.7161x over previous
import functools

import jax
import jax.numpy as jnp
from jax import lax
from jax.experimental import pallas as pl
from jax.experimental.pallas import tpu as pltpu
from jax.experimental.pallas import tpu_sc as plsc

MIN_VAL = 0
MAX_VAL = 999999
OUTPUT_DIM = 64
BATCH = 16384

_KERNEL_CACHE = {}


def _make_gather_kernel(batch, dim):
    info = plsc.get_sparse_core_info()
    num_cores, num_subcores, lanes = (
        info.num_cores,
        info.num_subcores,
        info.num_lanes,
    )
    num_workers = num_cores * num_subcores
    b_per_w = batch // num_workers
    mesh = plsc.VectorSubcoreMesh(core_axis_name="c", subcore_axis_name="s")

    @functools.partial(
        pl.kernel,
        mesh=mesh,
        out_type=jax.ShapeDtypeStruct((batch, dim), jnp.float32),
        scratch_types=[
            pltpu.VMEM((b_per_w,), jnp.int32),
            pltpu.VMEM((b_per_w, dim), jnp.float32),
            pltpu.SemaphoreType.DMA,
        ],
    )
    def gather_kernel(ints_hbm, table_hbm, out_hbm, idx_v, rows_v, sem):
        wid = lax.axis_index("s") * num_cores + lax.axis_index("c")
        base = wid * b_per_w
        pltpu.sync_copy(ints_hbm.at[pl.ds(base, b_per_w)], idx_v)
        for t in range(b_per_w // lanes):
            vec = idx_v[pl.ds(t * lanes, lanes)]
            vec = jnp.minimum(jnp.maximum(vec, MIN_VAL), MAX_VAL) - MIN_VAL
            for l in range(lanes):
                r = vec[l]
                pltpu.async_copy(
                    table_hbm.at[pl.ds(r, 1)],
                    rows_v.at[pl.ds(t * lanes + l, 1)],
                    sem,
                )
        pltpu.make_async_copy(
            table_hbm.at[pl.ds(0, b_per_w)], rows_v, sem
        ).wait()
        pltpu.sync_copy(rows_v, out_hbm.at[pl.ds(base, b_per_w)])

    return gather_kernel


def kernel(ints, table):
    batch = ints.shape[0]
    dim = table.shape[1]
    key = (batch, dim)
    if key not in _KERNEL_CACHE:
        _KERNEL_CACHE[key] = _make_gather_kernel(batch, dim)
    out = _KERNEL_CACHE[key](ints.astype(jnp.int32), table)
    mask = jnp.ones((batch, 1), dtype=jnp.float32)
    return out[:, None, :], mask


# speedup vs baseline: 1.7217x; 1.0032x over previous
import functools

import jax
import jax.numpy as jnp
from jax import lax
from jax.experimental import pallas as pl
from jax.experimental.pallas import tpu as pltpu
from jax.experimental.pallas import tpu_sc as plsc

MIN_VAL = 0
MAX_VAL = 999999
OUTPUT_DIM = 64
BATCH = 16384

_KERNEL_CACHE = {}


def _make_gather_kernel(batch, dim):
    info = plsc.get_sparse_core_info()
    num_cores, num_subcores, lanes = (
        info.num_cores,
        info.num_subcores,
        info.num_lanes,
    )
    num_workers = num_cores * num_subcores
    b_per_w = batch // num_workers
    mesh = plsc.VectorSubcoreMesh(core_axis_name="c", subcore_axis_name="s")

    @functools.partial(
        pl.kernel,
        mesh=mesh,
        compiler_params=pltpu.CompilerParams(use_tc_tiling_on_sc=True),
        out_type=jax.ShapeDtypeStruct((batch, dim), jnp.float32),
        scratch_types=[
            pltpu.VMEM((b_per_w,), jnp.int32),
            pltpu.VMEM((b_per_w, dim), jnp.float32),
            pltpu.SemaphoreType.DMA,
        ],
    )
    def gather_kernel(ints_hbm, table_hbm, out_hbm, idx_v, rows_v, sem):
        wid = lax.axis_index("s") * num_cores + lax.axis_index("c")
        base = wid * b_per_w
        pltpu.sync_copy(ints_hbm.at[pl.ds(base, b_per_w)], idx_v)
        for t in range(b_per_w // lanes):
            vec = idx_v[pl.ds(t * lanes, lanes)]
            vec = jnp.minimum(jnp.maximum(vec, MIN_VAL), MAX_VAL) - MIN_VAL
            for l in range(lanes):
                r = vec[l]
                pltpu.async_copy(
                    table_hbm.at[pl.ds(r, 1)],
                    rows_v.at[pl.ds(t * lanes + l, 1)],
                    sem,
                )
        pltpu.make_async_copy(
            table_hbm.at[pl.ds(0, b_per_w)], rows_v, sem
        ).wait()
        pltpu.sync_copy(rows_v, out_hbm.at[pl.ds(base, b_per_w)])

    return gather_kernel


def kernel(ints, table):
    batch = ints.shape[0]
    dim = table.shape[1]
    key = (batch, dim)
    if key not in _KERNEL_CACHE:
        _KERNEL_CACHE[key] = _make_gather_kernel(batch, dim)
    out = _KERNEL_CACHE[key](ints.astype(jnp.int32), table)
    mask = jnp.ones((batch, 1), dtype=jnp.float32)
    return out[:, None, :], mask


# speedup vs baseline: 2.5224x; 1.4651x over previous
import functools

import jax
import jax.numpy as jnp
from jax import lax
from jax.experimental import pallas as pl
from jax.experimental.pallas import tpu as pltpu
from jax.experimental.pallas import tpu_sc as plsc

MIN_VAL = 0
MAX_VAL = 999999
OUTPUT_DIM = 64
BATCH = 16384
SLAB = 128
NBUF = 8

_KERNEL_CACHE = {}


def _make_gather_kernel(batch, vocab, dim):
    info = plsc.get_sparse_core_info()
    num_cores, num_subcores, lanes = (
        info.num_cores,
        info.num_subcores,
        info.num_lanes,
    )
    num_workers = num_cores * num_subcores
    b_per_w = batch // num_workers
    n_chunk = b_per_w // lanes
    tail_start = (vocab // SLAB) * SLAB
    last_full = tail_start - SLAB
    mesh = plsc.VectorSubcoreMesh(core_axis_name="c", subcore_axis_name="s")

    @functools.partial(
        pl.kernel,
        mesh=mesh,
        compiler_params=pltpu.CompilerParams(
            use_tc_tiling_on_sc=True, needs_layout_passes=False
        ),
        out_type=jax.ShapeDtypeStruct((batch, SLAB), jnp.float32),
        scratch_types=[
            pltpu.VMEM((b_per_w,), jnp.int32),
            *[pltpu.VMEM((dim, SLAB), jnp.float32) for _ in range(NBUF)],
            pltpu.VMEM((lanes, SLAB), jnp.float32),
            pltpu.VMEM((SLAB, SLAB), jnp.float32),
            *[pltpu.SemaphoreType.DMA for _ in range(NBUF)],
            pltpu.SemaphoreType.DMA,
            pltpu.SemaphoreType.DMA,
        ],
    )
    def gather_kernel(ints_hbm, tableT_hbm, tail_hbm, out_hbm, idx_v, *refs):
        slabs = refs[:NBUF]
        ring = refs[NBUF]
        tail_v = refs[NBUF + 1]
        slab_sems = refs[NBUF + 2 : 2 * NBUF + 2]
        out_sem = refs[2 * NBUF + 2]
        tail_sem = refs[2 * NBUF + 3]

        wid = lax.axis_index("s") * num_cores + lax.axis_index("c")
        base = wid * b_per_w
        cvecs = [lax.iota(jnp.int32, lanes) + g * lanes for g in range(dim // lanes)]

        pltpu.sync_copy(ints_hbm.at[pl.ds(base, b_per_w)], idx_v)
        pltpu.async_copy(tail_hbm, tail_v, tail_sem)
        for t in range(n_chunk):
            sl = pl.ds(t * lanes, lanes)
            idx_v[sl] = (
                jnp.minimum(jnp.maximum(idx_v[sl], MIN_VAL), MAX_VAL) - MIN_VAL
            )

        def slab_start(r):
            return pl.multiple_of(
                jnp.minimum(lax.bitwise_and(r, -SLAB), last_full), SLAB
            )

        def fire(vec, l):
            r = vec[l % lanes]
            pltpu.async_copy(
                tableT_hbm.at[:, pl.ds(slab_start(r), SLAB)],
                slabs[l % NBUF],
                slab_sems[l % NBUF],
            )

        def extract(vec, c, l):
            s = l % NBUF
            pltpu.make_async_copy(
                tableT_hbm.at[:, pl.ds(0, SLAB)], slabs[s], slab_sems[s]
            ).wait()
            r = vec[l]
            lr = jnp.minimum(r - slab_start(r), SLAB - 1)
            lr_splat = jnp.full((lanes,), lr, jnp.int32)
            row = ring.at[l]
            for g in range(dim // lanes):
                row[pl.ds(g * lanes, lanes)] = plsc.load_gather(
                    slabs[s], [cvecs[g], lr_splat]
                )
            pltpu.async_copy(
                ring.at[pl.ds(l, 1), :],
                out_hbm.at[pl.ds(base + c * lanes + l, 1)],
                out_sem,
            )

        def out_drain():
            pltpu.make_async_copy(
                tableT_hbm.at[pl.ds(0, 1), pl.ds(0, SLAB)],
                ring.at[pl.ds(0, 1), :],
                out_sem,
            ).wait()

        def chunk_body(c, _):
            @pl.when(c > 0)
            def _():
                for _i in range(lanes):
                    out_drain()

            vec = idx_v[pl.ds(c * lanes, lanes)]
            for l in range(NBUF):
                fire(vec, l)
            for l in range(NBUF):
                extract(vec, c, l)
                fire(vec, l + NBUF)
            for l in range(NBUF, lanes):
                extract(vec, c, l)
            return 0

        lax.fori_loop(0, n_chunk, chunk_body, 0)
        for _i in range(lanes):
            out_drain()

        pltpu.make_async_copy(tail_hbm, tail_v, tail_sem).wait()

        def fixup_body(c, _):
            vec = idx_v[pl.ds(c * lanes, lanes)]
            n_tail = plsc.all_reduce_population_count(vec >= tail_start)

            @pl.when(n_tail[0] > 0)
            def _():
                for l in range(lanes):
                    r = vec[l]

                    @pl.when(r >= tail_start)
                    def _():
                        lr_splat = jnp.full((lanes,), r - tail_start, jnp.int32)
                        row = ring.at[0]
                        for g in range(dim // lanes):
                            row[pl.ds(g * lanes, lanes)] = plsc.load_gather(
                                tail_v, [lr_splat, cvecs[g]]
                            )
                        pltpu.async_copy(
                            ring.at[pl.ds(0, 1), :],
                            out_hbm.at[pl.ds(base + c * lanes + l, 1)],
                            tail_sem,
                        )
                        pltpu.make_async_copy(
                            tableT_hbm.at[pl.ds(0, 1), pl.ds(0, SLAB)],
                            ring.at[pl.ds(0, 1), :],
                            tail_sem,
                        ).wait()

            return 0

        lax.fori_loop(0, n_chunk, fixup_body, 0)

    return gather_kernel


def kernel(ints, table):
    batch = ints.shape[0]
    vocab, dim = table.shape
    key = (batch, vocab, dim)
    if key not in _KERNEL_CACHE:
        _KERNEL_CACHE[key] = _make_gather_kernel(batch, vocab, dim)
    tail_start = (vocab // SLAB) * SLAB
    tail_pad = (
        jnp.zeros((SLAB, SLAB), jnp.float32)
        .at[: vocab - tail_start, :dim]
        .set(table[tail_start:])
    )
    out = _KERNEL_CACHE[key](ints.astype(jnp.int32), table.T, tail_pad)
    mask = jnp.ones((batch, 1), dtype=jnp.float32)
    return out[:, :dim][:, None, :], mask
